# baseline (speedup 1.0000x reference)
import functools

import jax
import jax.numpy as jnp
from jax.experimental import pallas as pl
from jax.experimental.pallas import tpu as pltpu

_NUM_HIST = 4
_INTERVAL = 5
_THRESH = 2.0

_CHUNK = 512
_BITS = 16
_GROUP_B = 2
_SENTINEL = 2 ** 30


def _round_up(x, m):
    return (x + m - 1) // m * m


def _nearest_reward_kernel(plx_ref, ply_ref, plh_ref, plb_ref, pei_ref,
                           route_ref, ptx_ref, pty_ref, ppx_ref, ppy_ref,
                           ptb_ref, out_ref, key_ref, bnd_ref, *, group_sz,
                           nbatch):
    n = ptx_ref.shape[1]
    nwords = route_ref.shape[0]
    ppad = plb_ref.shape[1]

    ptx = ptx_ref[0]
    pty = pty_ref[0]
    ddx = ptx - ppx_ref[0]
    ddy = pty - ppy_ref[0]
    ptb = ptb_ref[0]

    @pl.when(pl.program_id(0) == 0)
    def _prologue():
        route_words = route_ref[:]

        def keystep(ck, _):
            s = pl.ds(ck * _CHUNK, _CHUNK)
            plb = plb_ref[:, s]
            ei = pei_ref[:, s]
            wrow = jax.lax.broadcasted_iota(jnp.int32, (nwords, _CHUNK), 0)
            hit = wrow == (ei >> 4)
            word = jnp.sum(jnp.where(hit, route_words, 0), axis=0,
                           keepdims=True)
            route = ((word >> (ei & (_BITS - 1))) & 1) > 0
            key_ref[:, s] = jnp.where(route, plb, _SENTINEL)
            return 0

        jax.lax.fori_loop(0, ppad // _CHUNK, keystep, 0)

        plb_all = plb_ref[:]
        bid = jax.lax.broadcasted_iota(jnp.int32, (nbatch, 1), 0)
        lt = jnp.sum((plb_all < bid).astype(jnp.int32), axis=1, keepdims=True)
        le = jnp.sum((plb_all <= bid).astype(jnp.int32), axis=1, keepdims=True)
        bnd_ref[:, 0:1] = lt
        bnd_ref[:, 1:2] = le

    brow = jax.lax.broadcasted_iota(jnp.int32, (nbatch, 1), 0)
    start = jnp.sum(jnp.where(brow == jnp.min(ptb), bnd_ref[:, 0:1], 0))
    end = jnp.sum(jnp.where(brow == jnp.max(ptb), bnd_ref[:, 1:2], 0))

    def body(ck, carry):
        bd, bv = carry
        s = pl.ds(ck * _CHUNK, _CHUNK)
        cx = plx_ref[:, s]
        cy = ply_ref[:, s]
        ph = plh_ref[:, s]
        key = key_ref[:, s]

        c = jnp.cos(ph)
        sn = jnp.sin(ph)
        dx = ptx - cx
        dy = pty - cy
        x = c * dx + sn * dy
        y = c * dy - sn * dx
        val = c * ddx + sn * ddy

        dist = jnp.abs(y) * 10.0 + jnp.abs(x) + jnp.where(x > 0, 1000.0, 0.0)
        dist = jnp.where(key == ptb, dist, jnp.inf)

        m = jnp.min(dist, axis=1, keepdims=True)
        lanepos = jax.lax.broadcasted_iota(jnp.int32, (n, _CHUNK), 1)
        eqm = dist == m
        first = jnp.min(jnp.where(eqm, lanepos, _CHUNK), axis=1, keepdims=True)
        sel = eqm & (lanepos == first)
        v = jnp.sum(jnp.where(sel, val, 0.0), axis=1, keepdims=True)

        upd = m < bd
        return jnp.where(upd, m, bd), jnp.where(upd, v, bv)

    init = (jnp.full((n, 1), jnp.inf, jnp.float32),
            jnp.zeros((n, 1), jnp.float32))
    bd, bv = jax.lax.fori_loop(start // _CHUNK, (end + _CHUNK - 1) // _CHUNK,
                               body, init)
    prog = jnp.where(jnp.isinf(bd), 0.0, bv)

    groups = 2 * _GROUP_B
    grow = jax.lax.broadcasted_iota(jnp.int32, (n, groups), 0) // group_sz
    gcol = jax.lax.broadcasted_iota(jnp.int32, (n, groups), 1)
    sums = jnp.sum(jnp.where(grow == gcol, prog, 0.0), axis=0, keepdims=True)

    progress = sums[:, :_GROUP_B]
    expert_progress = sums[:, _GROUP_B:]
    out_ref[0] = jnp.minimum(
        jnp.maximum(progress, _THRESH) / jnp.maximum(expert_progress, _THRESH),
        1.0)


def kernel(polyline_batch, polyline_position, polyline_heading,
           polyline_to_polygon_edge_index, polygon_on_route_mask,
           agent_ptr, agent_batch, agent_infer_position, agent_position):
    p = polyline_position.shape[0]
    npoly = polygon_on_route_mask.shape[0]
    b = agent_ptr.shape[0] - 1
    g = b // _GROUP_B

    ego_index = agent_ptr[:-1]
    infer = jnp.take(agent_infer_position, ego_index, axis=0)
    t = infer.shape[1] - _NUM_HIST
    ego_pos = infer[:, _NUM_HIST:]
    ego_pre = infer[:, _NUM_HIST - 1:-1]
    expert = jnp.take(agent_position, ego_index, axis=0)[:, ::_INTERVAL]
    exp_pre = expert[:, -t - 1:-1]
    exp_cur = expert[:, -t:]

    ego_c = ego_pos.reshape(g, _GROUP_B * t, 2)
    ego_p = ego_pre.reshape(g, _GROUP_B * t, 2)
    exp_c2 = exp_cur.reshape(g, _GROUP_B * t, 2)
    exp_p2 = exp_pre.reshape(g, _GROUP_B * t, 2)
    pts = jnp.concatenate([ego_c, exp_c2], axis=1)
    pre = jnp.concatenate([ego_p, exp_p2], axis=1)
    batch_ids = jnp.take(agent_batch, ego_index, axis=0).astype(jnp.int32)
    ptb = jnp.repeat(batch_ids, t).reshape(g, _GROUP_B * t)
    ptb = jnp.concatenate([ptb, ptb], axis=1)
    n = pts.shape[1]

    ppad = _round_up(p, _CHUNK)

    def rowvec(a, pad_value):
        return jnp.pad(a.reshape(1, p), ((0, 0), (0, ppad - p)),
                       constant_values=pad_value)

    plx = rowvec(polyline_position[:, 0], 0.0)
    ply = rowvec(polyline_position[:, 1], 0.0)
    plh = rowvec(polyline_heading, 0.0)
    plb = rowvec(polyline_batch.astype(jnp.int32), _SENTINEL)
    pei = rowvec(polyline_to_polygon_edge_index[1].astype(jnp.int32), 0)

    nwords = _round_up(npoly, _BITS * 128) // _BITS
    route_bits = jnp.pad(polygon_on_route_mask.astype(jnp.int32),
                         (0, nwords * _BITS - npoly)).reshape(nwords, _BITS)
    route = (route_bits @ (2 ** jnp.arange(_BITS, dtype=jnp.int32))).reshape(
        nwords, 1)

    full = lambda a: pl.BlockSpec(a.shape, lambda i: (0, 0))
    ptspec = pl.BlockSpec((1, n, 1), lambda i: (i, 0, 0))
    pt3 = lambda a: a.reshape(g, n, 1)
    out = pl.pallas_call(
        functools.partial(_nearest_reward_kernel, group_sz=t, nbatch=b),
        grid=(g,),
        in_specs=[full(plx), full(ply), full(plh), full(plb), full(pei),
                  full(route), ptspec, ptspec, ptspec, ptspec, ptspec],
        out_specs=pl.BlockSpec((1, 1, _GROUP_B), lambda i: (i, 0, 0)),
        out_shape=jax.ShapeDtypeStruct((g, 1, _GROUP_B), jnp.float32),
        scratch_shapes=[pltpu.VMEM((1, ppad), jnp.int32),
                        pltpu.VMEM((b, 2), jnp.int32)],
    )(plx, ply, plh, plb, pei, route,
      pt3(pts[:, :, 0]), pt3(pts[:, :, 1]),
      pt3(pre[:, :, 0]), pt3(pre[:, :, 1]), pt3(ptb))
    return out.reshape(b)


# speedup vs baseline: 1.2007x; 1.2007x over previous
import functools

import jax
import jax.numpy as jnp
from jax.experimental import pallas as pl
from jax.experimental.pallas import tpu as pltpu

_NUM_HIST = 4
_INTERVAL = 5
_THRESH = 2.0

_CHUNK = 1024
_BITS = 16
_GROUP_B = 4
_SENTINEL = 2 ** 30


def _round_up(x, m):
    return (x + m - 1) // m * m


def _nearest_reward_kernel(plx_ref, ply_ref, plh_ref, plb_ref, pei_ref,
                           route_ref, ptx_ref, pty_ref, ppx_ref, ppy_ref,
                           ptb_ref, out_ref, key_ref, bnd_ref, *, group_sz,
                           nbatch):
    n = ptx_ref.shape[1]
    nwords = route_ref.shape[0]
    ppad = plb_ref.shape[1]

    ptx = ptx_ref[0]
    pty = pty_ref[0]
    ddx = ptx - ppx_ref[0]
    ddy = pty - ppy_ref[0]
    ptb = ptb_ref[0]

    @pl.when(pl.program_id(0) == 0)
    def _prologue():
        route_words = route_ref[:]

        def keystep(ck, _):
            s = pl.ds(ck * _CHUNK, _CHUNK)
            plb = plb_ref[:, s]
            ei = pei_ref[:, s]
            wrow = jax.lax.broadcasted_iota(jnp.int32, (nwords, _CHUNK), 0)
            hit = wrow == (ei >> 4)
            word = jnp.sum(jnp.where(hit, route_words, 0), axis=0,
                           keepdims=True)
            route = ((word >> (ei & (_BITS - 1))) & 1) > 0
            key_ref[:, s] = jnp.where(route, plb, _SENTINEL)
            return 0

        jax.lax.fori_loop(0, ppad // _CHUNK, keystep, 0)

        plb_all = plb_ref[:]
        bid = jax.lax.broadcasted_iota(jnp.int32, (nbatch, 1), 0)
        lt = jnp.sum((plb_all < bid).astype(jnp.int32), axis=1, keepdims=True)
        le = jnp.sum((plb_all <= bid).astype(jnp.int32), axis=1, keepdims=True)
        bnd_ref[:, 0:1] = lt
        bnd_ref[:, 1:2] = le

    brow = jax.lax.broadcasted_iota(jnp.int32, (nbatch, 1), 0)
    start = jnp.sum(jnp.where(brow == jnp.min(ptb), bnd_ref[:, 0:1], 0))
    end = jnp.sum(jnp.where(brow == jnp.max(ptb), bnd_ref[:, 1:2], 0))

    def body(ck, carry):
        bd, bv = carry
        s = pl.ds(ck * _CHUNK, _CHUNK)
        cx = plx_ref[:, s]
        cy = ply_ref[:, s]
        ph = plh_ref[:, s]
        key = key_ref[:, s]

        c = jnp.cos(ph)
        sn = jnp.sin(ph)
        dx = ptx - cx
        dy = pty - cy
        x = c * dx + sn * dy
        y = c * dy - sn * dx
        val = c * ddx + sn * ddy

        dist = jnp.abs(y) * 10.0 + jnp.abs(x) + jnp.where(x > 0, 1000.0, 0.0)
        dist = jnp.where(key == ptb, dist, jnp.inf)

        m = jnp.min(dist, axis=1, keepdims=True)
        lanepos = jax.lax.broadcasted_iota(jnp.int32, (n, _CHUNK), 1)
        eqm = dist == m
        first = jnp.min(jnp.where(eqm, lanepos, _CHUNK), axis=1, keepdims=True)
        sel = eqm & (lanepos == first)
        v = jnp.sum(jnp.where(sel, val, 0.0), axis=1, keepdims=True)

        upd = m < bd
        return jnp.where(upd, m, bd), jnp.where(upd, v, bv)

    init = (jnp.full((n, 1), jnp.inf, jnp.float32),
            jnp.zeros((n, 1), jnp.float32))
    bd, bv = jax.lax.fori_loop(start // _CHUNK, (end + _CHUNK - 1) // _CHUNK,
                               body, init)
    prog = jnp.where(jnp.isinf(bd), 0.0, bv)

    groups = 2 * _GROUP_B
    grow = jax.lax.broadcasted_iota(jnp.int32, (n, groups), 0) // group_sz
    gcol = jax.lax.broadcasted_iota(jnp.int32, (n, groups), 1)
    sums = jnp.sum(jnp.where(grow == gcol, prog, 0.0), axis=0, keepdims=True)

    progress = sums[:, :_GROUP_B]
    expert_progress = sums[:, _GROUP_B:]
    out_ref[0] = jnp.minimum(
        jnp.maximum(progress, _THRESH) / jnp.maximum(expert_progress, _THRESH),
        1.0)


def kernel(polyline_batch, polyline_position, polyline_heading,
           polyline_to_polygon_edge_index, polygon_on_route_mask,
           agent_ptr, agent_batch, agent_infer_position, agent_position):
    p = polyline_position.shape[0]
    npoly = polygon_on_route_mask.shape[0]
    b = agent_ptr.shape[0] - 1
    g = b // _GROUP_B

    ego_index = agent_ptr[:-1]
    infer = jnp.take(agent_infer_position, ego_index, axis=0)
    t = infer.shape[1] - _NUM_HIST
    ego_pos = infer[:, _NUM_HIST:]
    ego_pre = infer[:, _NUM_HIST - 1:-1]
    expert = jnp.take(agent_position, ego_index, axis=0)[:, ::_INTERVAL]
    exp_pre = expert[:, -t - 1:-1]
    exp_cur = expert[:, -t:]

    ego_c = ego_pos.reshape(g, _GROUP_B * t, 2)
    ego_p = ego_pre.reshape(g, _GROUP_B * t, 2)
    exp_c2 = exp_cur.reshape(g, _GROUP_B * t, 2)
    exp_p2 = exp_pre.reshape(g, _GROUP_B * t, 2)
    pts = jnp.concatenate([ego_c, exp_c2], axis=1)
    pre = jnp.concatenate([ego_p, exp_p2], axis=1)
    batch_ids = jnp.take(agent_batch, ego_index, axis=0).astype(jnp.int32)
    ptb = jnp.repeat(batch_ids, t).reshape(g, _GROUP_B * t)
    ptb = jnp.concatenate([ptb, ptb], axis=1)
    n = pts.shape[1]

    ppad = _round_up(p, _CHUNK)

    def rowvec(a, pad_value):
        return jnp.pad(a.reshape(1, p), ((0, 0), (0, ppad - p)),
                       constant_values=pad_value)

    plx = rowvec(polyline_position[:, 0], 0.0)
    ply = rowvec(polyline_position[:, 1], 0.0)
    plh = rowvec(polyline_heading, 0.0)
    plb = rowvec(polyline_batch.astype(jnp.int32), _SENTINEL)
    pei = rowvec(polyline_to_polygon_edge_index[1].astype(jnp.int32), 0)

    nwords = _round_up(npoly, _BITS * 128) // _BITS
    route_bits = jnp.pad(polygon_on_route_mask.astype(jnp.int32),
                         (0, nwords * _BITS - npoly)).reshape(nwords, _BITS)
    route = (route_bits @ (2 ** jnp.arange(_BITS, dtype=jnp.int32))).reshape(
        nwords, 1)

    full = lambda a: pl.BlockSpec(a.shape, lambda i: (0, 0))
    ptspec = pl.BlockSpec((1, n, 1), lambda i: (i, 0, 0))
    pt3 = lambda a: a.reshape(g, n, 1)
    out = pl.pallas_call(
        functools.partial(_nearest_reward_kernel, group_sz=t, nbatch=b),
        grid=(g,),
        in_specs=[full(plx), full(ply), full(plh), full(plb), full(pei),
                  full(route), ptspec, ptspec, ptspec, ptspec, ptspec],
        out_specs=pl.BlockSpec((1, 1, _GROUP_B), lambda i: (i, 0, 0)),
        out_shape=jax.ShapeDtypeStruct((g, 1, _GROUP_B), jnp.float32),
        scratch_shapes=[pltpu.VMEM((1, ppad), jnp.int32),
                        pltpu.VMEM((b, 2), jnp.int32)],
    )(plx, ply, plh, plb, pei, route,
      pt3(pts[:, :, 0]), pt3(pts[:, :, 1]),
      pt3(pre[:, :, 0]), pt3(pre[:, :, 1]), pt3(ptb))
    return out.reshape(b)


# speedup vs baseline: 1.2204x; 1.0164x over previous
import functools

import jax
import jax.numpy as jnp
from jax.experimental import pallas as pl
from jax.experimental.pallas import tpu as pltpu

_NUM_HIST = 4
_INTERVAL = 5
_THRESH = 2.0

_CHUNK = 2048
_BITS = 16
_GROUP_B = 4


def _round_up(x, m):
    return (x + m - 1) // m * m


def _nearest_reward_kernel(plx_ref, ply_ref, plh_ref, plb_ref, pei_ref,
                           route_ref, ptx_ref, pty_ref, ppx_ref, ppy_ref,
                           ptb_ref, out_ref, *, group_sz):
    n = ptx_ref.shape[1]
    nwords = route_ref.shape[0]

    ptx = ptx_ref[0]
    pty = pty_ref[0]
    ddx = ptx - ppx_ref[0]
    ddy = pty - ppy_ref[0]
    ptb = ptb_ref[0]
    route_words = route_ref[:]

    plb_all = plb_ref[:]
    start = jnp.sum((plb_all < jnp.min(ptb)).astype(jnp.int32))
    end = jnp.sum((plb_all <= jnp.max(ptb)).astype(jnp.int32))

    def body(ck, carry):
        bd, bv = carry
        s = pl.ds(ck * _CHUNK, _CHUNK)
        cx = plx_ref[:, s]
        cy = ply_ref[:, s]
        ph = plh_ref[:, s]
        plb = plb_ref[:, s]
        ei = pei_ref[:, s]

        wrow = jax.lax.broadcasted_iota(jnp.int32, (nwords, _CHUNK), 0)
        hit = wrow == (ei >> 4)
        word = jnp.sum(jnp.where(hit, route_words, 0), axis=0, keepdims=True)
        route = ((word >> (ei & (_BITS - 1))) & 1) > 0

        c = jnp.cos(ph)
        sn = jnp.sin(ph)
        dx = ptx - cx
        dy = pty - cy
        x = c * dx + sn * dy
        y = c * dy - sn * dx
        val = c * ddx + sn * ddy

        mask = (plb == ptb) & route
        dist = jnp.abs(y) * 10.0 + jnp.abs(x) + jnp.where(x > 0, 1000.0, 0.0)
        dist = jnp.where(mask, dist, jnp.inf)

        m = jnp.min(dist, axis=1, keepdims=True)
        lanepos = jax.lax.broadcasted_iota(jnp.int32, (n, _CHUNK), 1)
        eqm = dist == m
        first = jnp.min(jnp.where(eqm, lanepos, _CHUNK), axis=1, keepdims=True)
        sel = eqm & (lanepos == first)
        v = jnp.sum(jnp.where(sel, val, 0.0), axis=1, keepdims=True)

        upd = m < bd
        return jnp.where(upd, m, bd), jnp.where(upd, v, bv)

    init = (jnp.full((n, 1), jnp.inf, jnp.float32),
            jnp.zeros((n, 1), jnp.float32))
    bd, bv = jax.lax.fori_loop(start // _CHUNK, (end + _CHUNK - 1) // _CHUNK,
                               body, init)
    prog = jnp.where(jnp.isinf(bd), 0.0, bv)

    groups = 2 * _GROUP_B
    grow = jax.lax.broadcasted_iota(jnp.int32, (n, groups), 0) // group_sz
    gcol = jax.lax.broadcasted_iota(jnp.int32, (n, groups), 1)
    sums = jnp.sum(jnp.where(grow == gcol, prog, 0.0), axis=0, keepdims=True)

    progress = sums[:, :_GROUP_B]
    expert_progress = sums[:, _GROUP_B:]
    out_ref[0] = jnp.minimum(
        jnp.maximum(progress, _THRESH) / jnp.maximum(expert_progress, _THRESH),
        1.0)


def kernel(polyline_batch, polyline_position, polyline_heading,
           polyline_to_polygon_edge_index, polygon_on_route_mask,
           agent_ptr, agent_batch, agent_infer_position, agent_position):
    p = polyline_position.shape[0]
    npoly = polygon_on_route_mask.shape[0]
    b = agent_ptr.shape[0] - 1
    g = b // _GROUP_B

    ego_index = agent_ptr[:-1]
    infer = jnp.take(agent_infer_position, ego_index, axis=0)
    t = infer.shape[1] - _NUM_HIST
    ego_pos = infer[:, _NUM_HIST:]
    ego_pre = infer[:, _NUM_HIST - 1:-1]
    expert = jnp.take(agent_position, ego_index, axis=0)[:, ::_INTERVAL]
    exp_pre = expert[:, -t - 1:-1]
    exp_cur = expert[:, -t:]

    ego_c = ego_pos.reshape(g, _GROUP_B * t, 2)
    ego_p = ego_pre.reshape(g, _GROUP_B * t, 2)
    exp_c2 = exp_cur.reshape(g, _GROUP_B * t, 2)
    exp_p2 = exp_pre.reshape(g, _GROUP_B * t, 2)
    pts = jnp.concatenate([ego_c, exp_c2], axis=1)
    pre = jnp.concatenate([ego_p, exp_p2], axis=1)
    batch_ids = jnp.take(agent_batch, ego_index, axis=0).astype(jnp.int32)
    ptb = jnp.repeat(batch_ids, t).reshape(g, _GROUP_B * t)
    ptb = jnp.concatenate([ptb, ptb], axis=1)
    n = pts.shape[1]

    ppad = _round_up(p, _CHUNK)

    def rowvec(a, pad_value):
        return jnp.pad(a.reshape(1, p), ((0, 0), (0, ppad - p)),
                       constant_values=pad_value)

    plx = rowvec(polyline_position[:, 0], 0.0)
    ply = rowvec(polyline_position[:, 1], 0.0)
    plh = rowvec(polyline_heading, 0.0)
    plb = rowvec(polyline_batch.astype(jnp.int32), 2 ** 30)
    pei = rowvec(polyline_to_polygon_edge_index[1].astype(jnp.int32), 0)

    nwords = _round_up(npoly, _BITS * 128) // _BITS
    route_bits = jnp.pad(polygon_on_route_mask.astype(jnp.int32),
                         (0, nwords * _BITS - npoly)).reshape(nwords, _BITS)
    route = (route_bits @ (2 ** jnp.arange(_BITS, dtype=jnp.int32))).reshape(
        nwords, 1)

    full = lambda a: pl.BlockSpec(a.shape, lambda i: (0, 0))
    ptspec = pl.BlockSpec((1, n, 1), lambda i: (i, 0, 0))
    pt3 = lambda a: a.reshape(g, n, 1)
    out = pl.pallas_call(
        functools.partial(_nearest_reward_kernel, group_sz=t),
        grid=(g,),
        in_specs=[full(plx), full(ply), full(plh), full(plb), full(pei),
                  full(route), ptspec, ptspec, ptspec, ptspec, ptspec],
        out_specs=pl.BlockSpec((1, 1, _GROUP_B), lambda i: (i, 0, 0)),
        out_shape=jax.ShapeDtypeStruct((g, 1, _GROUP_B), jnp.float32),
        compiler_params=pltpu.CompilerParams(
            dimension_semantics=("parallel",)),
    )(plx, ply, plh, plb, pei, route,
      pt3(pts[:, :, 0]), pt3(pts[:, :, 1]),
      pt3(pre[:, :, 0]), pt3(pre[:, :, 1]), pt3(ptb))
    return out.reshape(b)


# speedup vs baseline: 1.3063x; 1.0704x over previous
import functools

import jax
import jax.numpy as jnp
from jax.experimental import pallas as pl
from jax.experimental.pallas import tpu as pltpu

_NUM_HIST = 4
_INTERVAL = 5
_THRESH = 2.0

_CHUNK = 1024
_BITS = 16
_GROUP_B = 4


def _round_up(x, m):
    return (x + m - 1) // m * m


def _nearest_reward_kernel(plf_ref, pli_ref, route_ref, ptf_ref,
                           ptb_ref, out_ref, *, group_sz):
    n = ptf_ref.shape[1]
    nwords = route_ref.shape[0]

    ptf = ptf_ref[0]
    ptx = ptf[:, 0:1]
    pty = ptf[:, 1:2]
    ddx = ptx - ptf[:, 2:3]
    ddy = pty - ptf[:, 3:4]
    ptb = ptb_ref[0]
    route_words = route_ref[:]

    plb_all = pli_ref[0:1, :]
    start = jnp.sum((plb_all < jnp.min(ptb)).astype(jnp.int32))
    end = jnp.sum((plb_all <= jnp.max(ptb)).astype(jnp.int32))

    def body(ck, carry):
        bd, bv = carry
        s = pl.ds(ck * _CHUNK, _CHUNK)
        cx = plf_ref[0:1, s]
        cy = plf_ref[1:2, s]
        ph = plf_ref[2:3, s]
        plb = pli_ref[0:1, s]
        ei = pli_ref[1:2, s]

        wrow = jax.lax.broadcasted_iota(jnp.int32, (nwords, _CHUNK), 0)
        hit = wrow == (ei >> 4)
        word = jnp.sum(jnp.where(hit, route_words, 0), axis=0, keepdims=True)
        route = ((word >> (ei & (_BITS - 1))) & 1) > 0

        c = jnp.cos(ph)
        sn = jnp.sin(ph)
        dx = ptx - cx
        dy = pty - cy
        x = c * dx + sn * dy
        y = c * dy - sn * dx
        val = c * ddx + sn * ddy

        mask = (plb == ptb) & route
        dist = jnp.abs(y) * 10.0 + jnp.abs(x) + jnp.where(x > 0, 1000.0, 0.0)
        dist = jnp.where(mask, dist, jnp.inf)

        m = jnp.min(dist, axis=1, keepdims=True)
        lanepos = jax.lax.broadcasted_iota(jnp.int32, (n, _CHUNK), 1)
        eqm = dist == m
        first = jnp.min(jnp.where(eqm, lanepos, _CHUNK), axis=1, keepdims=True)
        sel = eqm & (lanepos == first)
        v = jnp.sum(jnp.where(sel, val, 0.0), axis=1, keepdims=True)

        upd = m < bd
        return jnp.where(upd, m, bd), jnp.where(upd, v, bv)

    init = (jnp.full((n, 1), jnp.inf, jnp.float32),
            jnp.zeros((n, 1), jnp.float32))
    bd, bv = jax.lax.fori_loop(start // _CHUNK, (end + _CHUNK - 1) // _CHUNK,
                               body, init)
    prog = jnp.where(jnp.isinf(bd), 0.0, bv)

    groups = 2 * _GROUP_B
    grow = jax.lax.broadcasted_iota(jnp.int32, (n, groups), 0) // group_sz
    gcol = jax.lax.broadcasted_iota(jnp.int32, (n, groups), 1)
    sums = jnp.sum(jnp.where(grow == gcol, prog, 0.0), axis=0, keepdims=True)

    progress = sums[:, :_GROUP_B]
    expert_progress = sums[:, _GROUP_B:]
    out_ref[0] = jnp.minimum(
        jnp.maximum(progress, _THRESH) / jnp.maximum(expert_progress, _THRESH),
        1.0)


def kernel(polyline_batch, polyline_position, polyline_heading,
           polyline_to_polygon_edge_index, polygon_on_route_mask,
           agent_ptr, agent_batch, agent_infer_position, agent_position):
    p = polyline_position.shape[0]
    npoly = polygon_on_route_mask.shape[0]
    b = agent_ptr.shape[0] - 1
    g = b // _GROUP_B

    ego_index = agent_ptr[:-1]
    infer = jnp.take(agent_infer_position, ego_index, axis=0)
    t = infer.shape[1] - _NUM_HIST
    ego_pos = infer[:, _NUM_HIST:]
    ego_pre = infer[:, _NUM_HIST - 1:-1]
    expert = jnp.take(agent_position, ego_index, axis=0)[:, ::_INTERVAL]
    exp_pre = expert[:, -t - 1:-1]
    exp_cur = expert[:, -t:]

    ego_c = ego_pos.reshape(g, _GROUP_B * t, 2)
    ego_p = ego_pre.reshape(g, _GROUP_B * t, 2)
    exp_c2 = exp_cur.reshape(g, _GROUP_B * t, 2)
    exp_p2 = exp_pre.reshape(g, _GROUP_B * t, 2)
    pts = jnp.concatenate([ego_c, exp_c2], axis=1)
    pre = jnp.concatenate([ego_p, exp_p2], axis=1)
    batch_ids = jnp.take(agent_batch, ego_index, axis=0).astype(jnp.int32)
    ptb = jnp.repeat(batch_ids, t).reshape(g, _GROUP_B * t)
    ptb = jnp.concatenate([ptb, ptb], axis=1)
    n = pts.shape[1]

    ppad = _round_up(p, _CHUNK)

    plf = jnp.pad(
        jnp.stack([polyline_position[:, 0], polyline_position[:, 1],
                   polyline_heading], axis=0),
        ((0, 0), (0, ppad - p)))
    pli = jnp.pad(
        jnp.stack([polyline_batch.astype(jnp.int32),
                   polyline_to_polygon_edge_index[1].astype(jnp.int32)],
                  axis=0),
        ((0, 0), (0, ppad - p)), constant_values=2 ** 30)

    nwords = _round_up(npoly, _BITS * 128) // _BITS
    route_bits = jnp.pad(polygon_on_route_mask.astype(jnp.int32),
                         (0, nwords * _BITS - npoly)).reshape(nwords, _BITS)
    route = (route_bits @ (2 ** jnp.arange(_BITS, dtype=jnp.int32))).reshape(
        nwords, 1)

    ptf = jnp.concatenate([pts, pre], axis=2)

    full = lambda a: pl.BlockSpec(a.shape, lambda i: (0, 0))
    out = pl.pallas_call(
        functools.partial(_nearest_reward_kernel, group_sz=t),
        grid=(g,),
        in_specs=[full(plf), full(pli), full(route),
                  pl.BlockSpec((1, n, 4), lambda i: (i, 0, 0)),
                  pl.BlockSpec((1, n, 1), lambda i: (i, 0, 0))],
        out_specs=pl.BlockSpec((1, 1, _GROUP_B), lambda i: (i, 0, 0)),
        out_shape=jax.ShapeDtypeStruct((g, 1, _GROUP_B), jnp.float32),
        compiler_params=pltpu.CompilerParams(
            dimension_semantics=("parallel",)),
    )(plf, pli, route, ptf, ptb.reshape(g, n, 1))
    return out.reshape(b)


# speedup vs baseline: 1.3345x; 1.0216x over previous
import functools

import jax
import jax.numpy as jnp
from jax.experimental import pallas as pl
from jax.experimental.pallas import tpu as pltpu

_NUM_HIST = 4
_INTERVAL = 5
_THRESH = 2.0

_CHUNK = 1280
_BITS = 16
_GROUP_B = 4


def _round_up(x, m):
    return (x + m - 1) // m * m


def _nearest_reward_kernel(plf_ref, pli_ref, route_ref, ptf_ref,
                           ptb_ref, out_ref, *, group_sz):
    n = ptf_ref.shape[1]
    nwords = route_ref.shape[0]

    ptf = ptf_ref[0]
    ptx = ptf[:, 0:1]
    pty = ptf[:, 1:2]
    ddx = ptx - ptf[:, 2:3]
    ddy = pty - ptf[:, 3:4]
    ptb = ptb_ref[0]
    route_words = route_ref[:]

    plb_all = pli_ref[0:1, :]
    start = jnp.sum((plb_all < jnp.min(ptb)).astype(jnp.int32))
    end = jnp.sum((plb_all <= jnp.max(ptb)).astype(jnp.int32))

    def body(ck, carry):
        bd, bv = carry
        s = pl.ds(ck * _CHUNK, _CHUNK)
        cx = plf_ref[0:1, s]
        cy = plf_ref[1:2, s]
        ph = plf_ref[2:3, s]
        plb = pli_ref[0:1, s]
        ei = pli_ref[1:2, s]

        wrow = jax.lax.broadcasted_iota(jnp.int32, (nwords, _CHUNK), 0)
        hit = wrow == (ei >> 4)
        word = jnp.sum(jnp.where(hit, route_words, 0), axis=0, keepdims=True)
        route = ((word >> (ei & (_BITS - 1))) & 1) > 0

        c = jnp.cos(ph)
        sn = jnp.sin(ph)
        dx = ptx - cx
        dy = pty - cy
        x = c * dx + sn * dy
        y = c * dy - sn * dx
        val = c * ddx + sn * ddy

        mask = (plb == ptb) & route
        dist = jnp.abs(y) * 10.0 + jnp.abs(x) + jnp.where(x > 0, 1000.0, 0.0)
        dist = jnp.where(mask, dist, jnp.inf)

        m = jnp.min(dist, axis=1, keepdims=True)
        lanepos = jax.lax.broadcasted_iota(jnp.int32, (n, _CHUNK), 1)
        eqm = dist == m
        first = jnp.min(jnp.where(eqm, lanepos, _CHUNK), axis=1, keepdims=True)
        sel = eqm & (lanepos == first)
        v = jnp.sum(jnp.where(sel, val, 0.0), axis=1, keepdims=True)

        upd = m < bd
        return jnp.where(upd, m, bd), jnp.where(upd, v, bv)

    init = (jnp.full((n, 1), jnp.inf, jnp.float32),
            jnp.zeros((n, 1), jnp.float32))
    bd, bv = jax.lax.fori_loop(start // _CHUNK, (end + _CHUNK - 1) // _CHUNK,
                               body, init)
    prog = jnp.where(jnp.isinf(bd), 0.0, bv)

    groups = 2 * _GROUP_B
    grow = jax.lax.broadcasted_iota(jnp.int32, (n, groups), 0) // group_sz
    gcol = jax.lax.broadcasted_iota(jnp.int32, (n, groups), 1)
    sums = jnp.sum(jnp.where(grow == gcol, prog, 0.0), axis=0, keepdims=True)

    progress = sums[:, :_GROUP_B]
    expert_progress = sums[:, _GROUP_B:]
    out_ref[0] = jnp.minimum(
        jnp.maximum(progress, _THRESH) / jnp.maximum(expert_progress, _THRESH),
        1.0)


def kernel(polyline_batch, polyline_position, polyline_heading,
           polyline_to_polygon_edge_index, polygon_on_route_mask,
           agent_ptr, agent_batch, agent_infer_position, agent_position):
    p = polyline_position.shape[0]
    npoly = polygon_on_route_mask.shape[0]
    b = agent_ptr.shape[0] - 1
    g = b // _GROUP_B

    ego_index = agent_ptr[:-1]
    infer = jnp.take(agent_infer_position, ego_index, axis=0)
    t = infer.shape[1] - _NUM_HIST
    ego_pos = infer[:, _NUM_HIST:]
    ego_pre = infer[:, _NUM_HIST - 1:-1]
    expert = jnp.take(agent_position, ego_index, axis=0)[:, ::_INTERVAL]
    exp_pre = expert[:, -t - 1:-1]
    exp_cur = expert[:, -t:]

    ego_c = ego_pos.reshape(g, _GROUP_B * t, 2)
    ego_p = ego_pre.reshape(g, _GROUP_B * t, 2)
    exp_c2 = exp_cur.reshape(g, _GROUP_B * t, 2)
    exp_p2 = exp_pre.reshape(g, _GROUP_B * t, 2)
    pts = jnp.concatenate([ego_c, exp_c2], axis=1)
    pre = jnp.concatenate([ego_p, exp_p2], axis=1)
    batch_ids = jnp.take(agent_batch, ego_index, axis=0).astype(jnp.int32)
    ptb = jnp.repeat(batch_ids, t).reshape(g, _GROUP_B * t)
    ptb = jnp.concatenate([ptb, ptb], axis=1)
    n = pts.shape[1]

    ppad = _round_up(p, _CHUNK)

    plf = jnp.pad(
        jnp.stack([polyline_position[:, 0], polyline_position[:, 1],
                   polyline_heading], axis=0),
        ((0, 0), (0, ppad - p)))
    pli = jnp.pad(
        jnp.stack([polyline_batch.astype(jnp.int32),
                   polyline_to_polygon_edge_index[1].astype(jnp.int32)],
                  axis=0),
        ((0, 0), (0, ppad - p)), constant_values=2 ** 30)

    nwords = _round_up(npoly, _BITS * 128) // _BITS
    route_bits = jnp.pad(polygon_on_route_mask.astype(jnp.int32),
                         (0, nwords * _BITS - npoly)).reshape(nwords, _BITS)
    route = (route_bits @ (2 ** jnp.arange(_BITS, dtype=jnp.int32))).reshape(
        nwords, 1)

    ptf = jnp.concatenate([pts, pre], axis=2)

    full = lambda a: pl.BlockSpec(a.shape, lambda i: (0, 0))
    out = pl.pallas_call(
        functools.partial(_nearest_reward_kernel, group_sz=t),
        grid=(g,),
        in_specs=[full(plf), full(pli), full(route),
                  pl.BlockSpec((1, n, 4), lambda i: (i, 0, 0)),
                  pl.BlockSpec((1, n, 1), lambda i: (i, 0, 0))],
        out_specs=pl.BlockSpec((1, 1, _GROUP_B), lambda i: (i, 0, 0)),
        out_shape=jax.ShapeDtypeStruct((g, 1, _GROUP_B), jnp.float32),
        compiler_params=pltpu.CompilerParams(
            dimension_semantics=("parallel",)),
    )(plf, pli, route, ptf, ptb.reshape(g, n, 1))
    return out.reshape(b)
